# Initial kernel scaffold; baseline (speedup 1.0000x reference)
import jax
import jax.numpy as jnp
from jax.experimental import pallas as pl


def kernel(self_tensor, k, dim, largest, sorted):
    raise NotImplementedError("write your pallas kernel here")


# speedup vs baseline: 1.3047x; 1.3047x over previous
import jax
import jax.numpy as jnp
from jax.experimental import pallas as pl

_R, _C = 128, 32768
_K = 64
_RB = 8


def _topk_body(x_ref, vals_ref, idx_ref):
    col = jax.lax.broadcasted_iota(jnp.int32, (_RB, _C), 1)
    lane = jax.lax.broadcasted_iota(jnp.int32, (_RB, _K), 1)

    def step(t, carry):
        x, vals, idxs = carry
        m = jnp.max(x, axis=1, keepdims=True)
        cand = jnp.where(x == m, col, jnp.int32(_C))
        i = jnp.min(cand, axis=1, keepdims=True)
        vals = jnp.where(lane == t, m, vals)
        idxs = jnp.where(lane == t, i, idxs)
        x = jnp.where(col == i, -jnp.inf, x)
        return x, vals, idxs

    zero_v = jnp.zeros((_RB, _K), jnp.float32)
    zero_i = jnp.zeros((_RB, _K), jnp.int32)
    _, vals, idxs = jax.lax.fori_loop(
        0, _K, step, (x_ref[...], zero_v, zero_i), unroll=False)
    vals_ref[...] = vals
    idx_ref[...] = idxs


def kernel(self_tensor, k, dim, largest, sorted):
    del k, dim, largest, sorted
    vals, idxs = pl.pallas_call(
        _topk_body,
        grid=(_R // _RB,),
        in_specs=[pl.BlockSpec((_RB, _C), lambda i: (i, 0))],
        out_specs=[
            pl.BlockSpec((_RB, _K), lambda i: (i, 0)),
            pl.BlockSpec((_RB, _K), lambda i: (i, 0)),
        ],
        out_shape=[
            jax.ShapeDtypeStruct((_R, _K), jnp.float32),
            jax.ShapeDtypeStruct((_R, _K), jnp.int32),
        ],
    )(self_tensor)
    return (vals, idxs)


# speedup vs baseline: 4.4292x; 3.3947x over previous
import jax
import jax.numpy as jnp
from jax import lax
from jax.experimental import pallas as pl
from jax.experimental.pallas import tpu as pltpu
from jax.experimental.pallas import tpu_sc as plsc

_R, _C = 128, 32768
_K = 64
_L = 16
_NVREG = _C // _L
_CAP = 288
_TRIG = 256
_HARD = 192
_NW = 32
_ROWS_PER_W = _R // _NW
_NEG = float("-inf")
_BIGI = 0x7FFFFFF0


def _lane(vec, i):
    return jnp.squeeze(lax.slice(vec, (i,), (i + 1,)))


def _select64(regv, regi, oref, outv, outi):
    nv = (oref[0] + _L - 1) // _L
    lanes = lax.iota(jnp.int32, _L)

    def round_body(t, _):
        def scanv(j, bc):
            bv, bi = bc
            v = regv[pl.ds(j * _L, _L)]
            i = regi[pl.ds(j * _L, _L)]
            better = (v > bv) | ((v == bv) & (i < bi))
            return (jnp.where(better, v, bv), jnp.where(better, i, bi))

        bv, bi = lax.fori_loop(
            0, nv, scanv,
            (jnp.full((_L,), _NEG, jnp.float32),
             jnp.full((_L,), _BIGI, jnp.int32)))
        mv = _lane(lax.sort(bv), _L - 1)
        wi = jnp.where(bv == mv, bi, jnp.int32(_BIGI))
        mi = _lane(lax.sort(wi), 0)
        slot = pl.ds((t // _L) * _L, _L)
        lv = t % _L
        outv[slot] = jnp.where(lanes == lv, mv, outv[slot])
        outi[slot] = jnp.where(lanes == lv, mi, outi[slot])

        def clearv(j, _c):
            s = pl.ds(j * _L, _L)
            v = regv[s]
            hit = (v == mv) & (regi[s] == mi)
            regv[s] = jnp.where(hit, _NEG, v)
            return 0

        lax.fori_loop(0, nv, clearv, 0)
        return 0

    lax.fori_loop(0, _K, round_body, 0)


def _sc_body(x_hbm, out_v_hbm, out_i_hbm,
             rowbuf, regv, regi, outv, outi, thref, oref):
    wid = lax.axis_index("s") * 2 + lax.axis_index("c")
    lanes = lax.iota(jnp.int32, _L)

    def do_row(rj, _):
        row = wid * _ROWS_PER_W + rj
        pltpu.sync_copy(x_hbm.at[row], rowbuf)
        thref[0] = jnp.float32(_NEG)
        oref[0] = jnp.int32(0)

        def init(j, _c):
            s = pl.ds(j * _L, _L)
            regv[s] = jnp.full((_L,), _NEG, jnp.float32)
            regi[s] = jnp.full((_L,), _BIGI, jnp.int32)
            return 0

        lax.fori_loop(0, _CAP // _L, init, 0)

        def hard_prune():
            _select64(regv, regi, oref, outv, outi)
            def put(j, _c):
                s = pl.ds(j * _L, _L)
                regv[s] = outv[s]
                regi[s] = outi[s]
                return 0
            lax.fori_loop(0, _K // _L, put, 0)

            def fill(j, _c):
                s = pl.ds(j * _L, _L)
                regv[s] = jnp.full((_L,), _NEG, jnp.float32)
                regi[s] = jnp.full((_L,), _BIGI, jnp.int32)
                return 0
            lax.fori_loop(_K // _L, _CAP // _L, fill, 0)
            oref[0] = jnp.int32(_K)
            tail = outv[pl.ds(_K - _L, _L)]
            thref[0] = jnp.maximum(thref[0], _lane(tail, _L - 1))

        def soft_prune():
            def tstep(j, tv):
                sv = lax.sort(regv[pl.ds(j * _L, _L)])
                return jnp.minimum(tv, sv)

            tv = lax.fori_loop(0, _TRIG // _L, tstep,
                               jnp.full((_L,), float("inf"), jnp.float32))
            t = _lane(tv, _L - 4)

            def cstep(j, o2):
                s = pl.ds(j * _L, _L)
                v = regv[s]
                i = regi[s]
                m = v >= t
                plsc.store_compressed(regv.at[pl.ds(o2, _L)], v, mask=m)
                plsc.store_compressed(regi.at[pl.ds(o2, _L)], i, mask=m)
                pc = plsc.all_reduce_population_count(m)
                return o2 + _lane(pc, 0)

            o2 = lax.fori_loop(0, _CAP // _L, cstep, jnp.int32(0))

            def rstep(j, _c):
                s = pl.ds(j * _L, _L)
                keep = (j * _L + lanes) < o2
                regv[s] = jnp.where(keep, regv[s], _NEG)
                regi[s] = jnp.where(keep, regi[s], _BIGI)
                return 0

            lax.fori_loop(0, _CAP // _L, rstep, 0)
            thref[0] = jnp.maximum(thref[0], t)
            oref[0] = o2

            @pl.when(o2 >= _HARD)
            def _():
                hard_prune()

        def scanstep(j, _c):
            v = rowbuf[pl.ds(j * _L, _L)]
            m = v >= thref[0]
            pc = plsc.all_reduce_population_count(m)
            cnt = _lane(pc, 0)
            o = oref[0]
            iv = j * _L + lanes
            plsc.store_compressed(regv.at[pl.ds(o, _L)], v, mask=m)
            plsc.store_compressed(regi.at[pl.ds(o, _L)], iv, mask=m)
            oref[0] = o + cnt

            @pl.when(o + cnt >= _TRIG)
            def _():
                soft_prune()

            return 0

        lax.fori_loop(0, _NVREG, scanstep, 0)

        _select64(regv, regi, oref, outv, outi)
        pltpu.sync_copy(outv, out_v_hbm.at[row])
        pltpu.sync_copy(outi, out_i_hbm.at[row])
        return 0

    lax.fori_loop(0, _ROWS_PER_W, do_row, 0)


@jax.jit
def _sc_topk(x):
    mesh = plsc.VectorSubcoreMesh(core_axis_name="c", subcore_axis_name="s")
    fn = pl.kernel(
        _sc_body,
        mesh=mesh,
        compiler_params=pltpu.CompilerParams(needs_layout_passes=False),
        out_type=[
            jax.ShapeDtypeStruct((_R, _K), jnp.float32),
            jax.ShapeDtypeStruct((_R, _K), jnp.int32),
        ],
        scratch_types=[
            pltpu.VMEM((_C,), jnp.float32),
            pltpu.VMEM((_CAP,), jnp.float32),
            pltpu.VMEM((_CAP,), jnp.int32),
            pltpu.VMEM((_K,), jnp.float32),
            pltpu.VMEM((_K,), jnp.int32),
            pltpu.SMEM((1,), jnp.float32),
            pltpu.SMEM((1,), jnp.int32),
        ],
    )
    return fn(x)


def kernel(self_tensor, k, dim, largest, sorted):
    del k, dim, largest, sorted
    vals, idxs = _sc_topk(self_tensor)
    return (vals, idxs)


# speedup vs baseline: 5.6224x; 1.2694x over previous
import jax
import jax.numpy as jnp
from jax import lax
from jax.experimental import pallas as pl
from jax.experimental.pallas import tpu as pltpu
from jax.experimental.pallas import tpu_sc as plsc

_R, _C = 128, 32768
_K = 64
_L = 16
_NVREG = _C // _L
_BLK = 8
_CAP = 416
_TRIG = 256
_HARD = 192
_NW = 32
_ROWS_PER_W = _R // _NW
_NEG = float("-inf")
_BIGI = 0x7FFFFFF0


def _lane(vec, i):
    return jnp.squeeze(lax.slice(vec, (i,), (i + 1,)))


def _select64(regv, regi, oref, outv, outi):
    nv = (oref[0] + _L - 1) // _L
    lanes = lax.iota(jnp.int32, _L)

    def round_body(t, _):
        def scanv(j, bc):
            bv, bi = bc
            v = regv[pl.ds(j * _L, _L)]
            i = regi[pl.ds(j * _L, _L)]
            better = (v > bv) | ((v == bv) & (i < bi))
            return (jnp.where(better, v, bv), jnp.where(better, i, bi))

        bv, bi = lax.fori_loop(
            0, nv, scanv,
            (jnp.full((_L,), _NEG, jnp.float32),
             jnp.full((_L,), _BIGI, jnp.int32)))
        mv = _lane(lax.sort(bv), _L - 1)
        wi = jnp.where(bv == mv, bi, jnp.int32(_BIGI))
        mi = _lane(lax.sort(wi), 0)
        slot = pl.ds((t // _L) * _L, _L)
        lv = t % _L
        outv[slot] = jnp.where(lanes == lv, mv, outv[slot])
        outi[slot] = jnp.where(lanes == lv, mi, outi[slot])

        def clearv(j, _c):
            s = pl.ds(j * _L, _L)
            v = regv[s]
            hit = (v == mv) & (regi[s] == mi)
            regv[s] = jnp.where(hit, _NEG, v)
            return 0

        lax.fori_loop(0, nv, clearv, 0)
        return 0

    lax.fori_loop(0, _K, round_body, 0)


def _sc_body(x_hbm, out_v_hbm, out_i_hbm,
             rowbuf, regv, regi, outv, outi, thref, oref):
    wid = lax.axis_index("s") * 2 + lax.axis_index("c")
    lanes = lax.iota(jnp.int32, _L)

    def do_row(rj, _):
        row = wid * _ROWS_PER_W + rj
        pltpu.sync_copy(x_hbm.at[row], rowbuf)
        thref[0] = jnp.float32(_NEG)
        oref[0] = jnp.int32(0)

        def init(j, _c):
            s = pl.ds(j * _L, _L)
            regv[s] = jnp.full((_L,), _NEG, jnp.float32)
            regi[s] = jnp.full((_L,), _BIGI, jnp.int32)
            return 0

        lax.fori_loop(0, _CAP // _L, init, 0)

        def hard_prune():
            _select64(regv, regi, oref, outv, outi)
            def put(j, _c):
                s = pl.ds(j * _L, _L)
                regv[s] = outv[s]
                regi[s] = outi[s]
                return 0
            lax.fori_loop(0, _K // _L, put, 0)

            def fill(j, _c):
                s = pl.ds(j * _L, _L)
                regv[s] = jnp.full((_L,), _NEG, jnp.float32)
                regi[s] = jnp.full((_L,), _BIGI, jnp.int32)
                return 0
            lax.fori_loop(_K // _L, _CAP // _L, fill, 0)
            oref[0] = jnp.int32(_K)
            tail = outv[pl.ds(_K - _L, _L)]
            thref[0] = jnp.maximum(thref[0], _lane(tail, _L - 1))

        def soft_prune():
            o = oref[0]
            nfull = o // _L
            c = (_K + nfull - 1) // nfull
            nv2 = (o + _L - 1) // _L

            def tstep(j, tv):
                sv = lax.sort(regv[pl.ds(j * _L, _L)])
                return jnp.minimum(tv, sv)

            tv = lax.fori_loop(0, nfull, tstep,
                               jnp.full((_L,), float("inf"), jnp.float32))
            tl = jnp.where(lanes == (_L - c), tv, float("inf"))
            t = _lane(lax.sort(tl), 0)

            def cstep(j, o2):
                s = pl.ds(j * _L, _L)
                v = regv[s]
                i = regi[s]
                m = v >= t
                plsc.store_compressed(regv.at[pl.ds(o2, _L)], v, mask=m)
                plsc.store_compressed(regi.at[pl.ds(o2, _L)], i, mask=m)
                pc = plsc.all_reduce_population_count(m)
                return o2 + _lane(pc, 0)

            o2 = lax.fori_loop(0, nv2, cstep, jnp.int32(0))

            def rstep(j, _c):
                s = pl.ds(j * _L, _L)
                keep = (j * _L + lanes) < o2
                regv[s] = jnp.where(keep, regv[s], _NEG)
                regi[s] = jnp.where(keep, regi[s], _BIGI)
                return 0

            lax.fori_loop(0, nv2, rstep, 0)
            thref[0] = jnp.maximum(thref[0], t)
            oref[0] = o2

            @pl.when(o2 >= _HARD)
            def _():
                hard_prune()

        def scanblock(b, _c):
            base = b * (_BLK * _L)
            mx = rowbuf[pl.ds(base, _L)]
            for u in range(1, _BLK):
                mx = jnp.maximum(mx, rowbuf[pl.ds(base + u * _L, _L)])
            anym = mx >= thref[0]
            pc = plsc.all_reduce_population_count(anym)

            @pl.when(_lane(pc, 0) > 0)
            def _():
                for u in range(_BLK):
                    off = base + u * _L
                    v = rowbuf[pl.ds(off, _L)]
                    m = v >= thref[0]
                    pcc = plsc.all_reduce_population_count(m)
                    o = oref[0]
                    plsc.store_compressed(regv.at[pl.ds(o, _L)], v, mask=m)
                    plsc.store_compressed(regi.at[pl.ds(o, _L)],
                                          off + lanes, mask=m)
                    oref[0] = o + _lane(pcc, 0)

                @pl.when(oref[0] >= _TRIG)
                def _():
                    soft_prune()

            return 0

        lax.fori_loop(0, _NVREG // _BLK, scanblock, 0)

        @pl.when(oref[0] >= 128)
        def _():
            soft_prune()

        _select64(regv, regi, oref, outv, outi)
        pltpu.sync_copy(outv, out_v_hbm.at[row])
        pltpu.sync_copy(outi, out_i_hbm.at[row])
        return 0

    lax.fori_loop(0, _ROWS_PER_W, do_row, 0)


@jax.jit
def _sc_topk(x):
    mesh = plsc.VectorSubcoreMesh(core_axis_name="c", subcore_axis_name="s")
    fn = pl.kernel(
        _sc_body,
        mesh=mesh,
        compiler_params=pltpu.CompilerParams(needs_layout_passes=False),
        out_type=[
            jax.ShapeDtypeStruct((_R, _K), jnp.float32),
            jax.ShapeDtypeStruct((_R, _K), jnp.int32),
        ],
        scratch_types=[
            pltpu.VMEM((_C,), jnp.float32),
            pltpu.VMEM((_CAP,), jnp.float32),
            pltpu.VMEM((_CAP,), jnp.int32),
            pltpu.VMEM((_K,), jnp.float32),
            pltpu.VMEM((_K,), jnp.int32),
            pltpu.SMEM((1,), jnp.float32),
            pltpu.SMEM((1,), jnp.int32),
        ],
    )
    return fn(x)


def kernel(self_tensor, k, dim, largest, sorted):
    del k, dim, largest, sorted
    vals, idxs = _sc_topk(self_tensor)
    return (vals, idxs)
